# Initial kernel scaffold; baseline (speedup 1.0000x reference)
import jax
import jax.numpy as jnp
from jax.experimental import pallas as pl


def kernel(x, ei, Wq, bq, Wk, bk, Wv, bv, Ws, bs):
    raise NotImplementedError("write your pallas kernel here")


# speedup vs baseline: 4.8313x; 4.8313x over previous
import functools

import jax
import jax.numpy as jnp
from jax import lax
from jax.experimental import pallas as pl
from jax.experimental.pallas import tpu as pltpu
from jax.experimental.pallas import tpu_sc as plsc

_NC = 2
_NS = 16
_L = 16
_NW = _NC * _NS
_SUB = 40
_C = 16


def _cdiv(a, b):
    return (a + b - 1) // b


def _proj_body(x_ref, wq_ref, bq_ref, wk_ref, bk_ref, wv_ref, bv_ref,
               ws_ref, bs_ref, q_ref, k_ref, v_ref, s_ref):
    xb = x_ref[...]
    q_ref[...] = jnp.dot(xb, wq_ref[...], preferred_element_type=jnp.float32) + bq_ref[...]
    k_ref[...] = jnp.dot(xb, wk_ref[...], preferred_element_type=jnp.float32) + bk_ref[...]
    v_ref[...] = jnp.dot(xb, wv_ref[...], preferred_element_type=jnp.float32) + bv_ref[...]
    s_ref[...] = xb + jnp.dot(xb, ws_ref[...], preferred_element_type=jnp.float32) + bs_ref[...]


def _edge_body(nper, nsub, H, D,
               q_hbm, k_hbm, v_hbm, skip_hbm, es_hbm, ed_hbm, rp_hbm, out_hbm,
               agg, den, outb, skipb, rpbv, esb, edb, edsx, qrows, krows,
               vrows, dotbuf, semq, semk, semv):
    HD = H * D
    TD = D // _L
    cid = lax.axis_index("c")
    sid = lax.axis_index("s")
    wid = sid * _NC + cid
    nbase = wid * nper
    inv_sqrt = 1.0 / float(D) ** 0.5
    lane = lax.iota(jnp.int32, _L)
    zeros16 = jnp.zeros((_L,), jnp.float32)
    gidx = [lane * _L + l for l in range(_L)]

    def sub_body(sub, _):
        m0 = nbase + sub * _SUB

        def z_body(i, _):
            base = i * 128
            for t in range(8):
                agg[pl.ds(base + t * _L, _L)] = zeros16
            return 0
        lax.fori_loop(0, (_SUB * HD) // 128, z_body, 0)

        def zd_body(i, _):
            den[pl.ds(i * _L, _L)] = zeros16
            return 0
        lax.fori_loop(0, _SUB, zd_body, 0)

        pltpu.sync_copy(rp_hbm.at[pl.ds(m0, 4 * _L)], rpbv)
        pltpu.sync_copy(skip_hbm.at[pl.ds(m0 * D, _SUB * D)], skipb)
        e_lo = rpbv[pl.ds(0, _L)][0]
        e_hi = rpbv[pl.ds(_SUB, _L)][0]
        c0 = e_lo // _C
        c1 = (e_hi + (_C - 1)) // _C

        def chunk_body(ci, _):
            ebase = ci * _C
            pltpu.sync_copy(es_hbm.at[pl.ds(ebase, _C)], esb)
            pltpu.sync_copy(ed_hbm.at[pl.ds(ebase, _C)], edb)
            pltpu.sync_copy(ed_hbm.at[pl.ds(ebase, 2 * _C)], edsx)
            cq = pltpu.async_copy(q_hbm.at[edb], qrows, semq)
            ck = pltpu.async_copy(k_hbm.at[esb], krows, semk)
            cv = pltpu.async_copy(v_hbm.at[esb], vrows, semv)
            cq.wait()
            ck.wait()
            cv.wait()

            def edge_body(j, _):
                d_e = edsx[pl.ds(j, _L)][0]
                ok = (d_e >= m0) & (d_e < m0 + _SUB) & (ebase + j < e_hi)

                @pl.when(ok)
                def _():
                    local = d_e - m0
                    for h in range(H):
                        acc = zeros16
                        for t in range(TD):
                            off = h * D + t * _L
                            acc = acc + (qrows[j, pl.ds(off, _L)]
                                         * krows[j, pl.ds(off, _L)])
                        dotbuf[pl.ds(h * _L, _L)] = acc
                    lvec = zeros16
                    for l in range(_L):
                        lvec = lvec + plsc.load_gather(dotbuf, [gidx[l]])
                    ex = jnp.exp(lvec * inv_sqrt)
                    ex = jnp.where(lane < H, ex, 0.0)
                    plsc.addupdate(den.at[pl.ds(local * _L, _L)], ex)
                    for h in range(H):
                        eh = ex[h]
                        for t in range(TD):
                            off = h * D + t * _L
                            plsc.addupdate(agg.at[pl.ds(local * HD + off, _L)],
                                           eh * vrows[j, pl.ds(off, _L)])
                return 0
            lax.fori_loop(0, _C, edge_body, 0)
            return 0
        lax.fori_loop(c0, c1, chunk_body, 0)

        def fin_body(n, _):
            dvec = den[pl.ds(n * _L, _L)]
            inv = (1.0 / float(H)) / (dvec + 1e-16)
            for t in range(TD):
                o = skipb[pl.ds(n * D + t * _L, _L)]
                for h in range(H):
                    o = o + inv[h] * agg[pl.ds(n * HD + h * D + t * _L, _L)]
                outb[pl.ds(n * D + t * _L, _L)] = o
            return 0
        lax.fori_loop(0, _SUB, fin_body, 0)
        pltpu.sync_copy(outb, out_hbm.at[pl.ds(m0 * D, _SUB * D)])
        return 0
    lax.fori_loop(0, nsub, sub_body, 0)


def kernel(x, ei, Wq, bq, Wk, bk, Wv, bv, Ws, bs):
    N, D = x.shape
    HD = Wq.shape[1]
    H = HD // D
    E = ei.shape[1]
    nper = _cdiv(N, _NW * _SUB) * _SUB
    npad = nper * _NW
    nsub = nper // _SUB

    xp = jnp.pad(x, ((0, npad - N), (0, 0)))
    bn = 1280
    while npad % bn != 0:
        bn //= 2
    q_p, k_p, v_p, skip_p = pl.pallas_call(
        _proj_body,
        grid=(npad // bn,),
        in_specs=[
            pl.BlockSpec((bn, D), lambda i: (i, 0)),
            pl.BlockSpec((D, HD), lambda i: (0, 0)),
            pl.BlockSpec((1, HD), lambda i: (0, 0)),
            pl.BlockSpec((D, HD), lambda i: (0, 0)),
            pl.BlockSpec((1, HD), lambda i: (0, 0)),
            pl.BlockSpec((D, HD), lambda i: (0, 0)),
            pl.BlockSpec((1, HD), lambda i: (0, 0)),
            pl.BlockSpec((D, D), lambda i: (0, 0)),
            pl.BlockSpec((1, D), lambda i: (0, 0)),
        ],
        out_specs=[
            pl.BlockSpec((bn, HD), lambda i: (i, 0)),
            pl.BlockSpec((bn, HD), lambda i: (i, 0)),
            pl.BlockSpec((bn, HD), lambda i: (i, 0)),
            pl.BlockSpec((bn, D), lambda i: (i, 0)),
        ],
        out_shape=[
            jax.ShapeDtypeStruct((npad, HD), jnp.float32),
            jax.ShapeDtypeStruct((npad, HD), jnp.float32),
            jax.ShapeDtypeStruct((npad, HD), jnp.float32),
            jax.ShapeDtypeStruct((npad, D), jnp.float32),
        ],
    )(xp, Wq, bq.reshape(1, HD), Wk, bk.reshape(1, HD),
      Wv, bv.reshape(1, HD), Ws, bs.reshape(1, D))

    src = ei[0].astype(jnp.int32)
    dst = ei[1].astype(jnp.int32)
    order = jnp.argsort(dst)
    ed_s = dst[order]
    es_s = src[order]
    ep = _cdiv(E, _C) * _C + 3 * _C
    es_pad = jnp.pad(es_s, (0, ep - E))
    ed_pad = jnp.pad(ed_s, (0, ep - E), constant_values=npad - 1)
    rp = jnp.searchsorted(
        ed_s, jnp.arange(npad + 4 * _L, dtype=jnp.int32), side="left"
    ).astype(jnp.int32)

    mesh = plsc.VectorSubcoreMesh(core_axis_name="c", subcore_axis_name="s",
                                  num_cores=_NC, num_subcores=_NS)
    edge_call = functools.partial(
        pl.kernel,
        out_type=jax.ShapeDtypeStruct((npad * D,), jnp.float32),
        mesh=mesh,
        compiler_params=pltpu.CompilerParams(needs_layout_passes=False),
        scratch_types=[
            pltpu.VMEM((_SUB * HD,), jnp.float32),
            pltpu.VMEM((_SUB * _L,), jnp.float32),
            pltpu.VMEM((_SUB * D,), jnp.float32),
            pltpu.VMEM((_SUB * D,), jnp.float32),
            pltpu.VMEM((4 * _L,), jnp.int32),
            pltpu.VMEM((_C,), jnp.int32),
            pltpu.VMEM((_C,), jnp.int32),
            pltpu.VMEM((2 * _C,), jnp.int32),
            pltpu.VMEM((_C, HD), jnp.float32),
            pltpu.VMEM((_C, HD), jnp.float32),
            pltpu.VMEM((_C, HD), jnp.float32),
            pltpu.VMEM((_L * _L,), jnp.float32),
            pltpu.SemaphoreType.DMA,
            pltpu.SemaphoreType.DMA,
            pltpu.SemaphoreType.DMA,
        ],
    )(functools.partial(_edge_body, nper, nsub, H, D))
    out_flat = edge_call(q_p, k_p, v_p, skip_p.reshape(npad * D),
                         es_pad, ed_pad, rp)
    return out_flat.reshape(npad, D)[:N]


# speedup vs baseline: 6.8221x; 1.4121x over previous
import functools

import jax
import jax.numpy as jnp
from jax import lax
from jax.experimental import pallas as pl
from jax.experimental.pallas import tpu as pltpu
from jax.experimental.pallas import tpu_sc as plsc

_NC = 2
_NS = 16
_L = 16
_NW = _NC * _NS
_SUB = 40
_C = 8


def _cdiv(a, b):
    return (a + b - 1) // b


def _proj_body(x_ref, wq_ref, bq_ref, wk_ref, bk_ref, wv_ref, bv_ref,
               ws_ref, bs_ref, q_ref, k_ref, v_ref, s_ref):
    xb = x_ref[...]
    q_ref[...] = jnp.dot(xb, wq_ref[...], preferred_element_type=jnp.float32) + bq_ref[...]
    k_ref[...] = jnp.dot(xb, wk_ref[...], preferred_element_type=jnp.float32) + bk_ref[...]
    v_ref[...] = jnp.dot(xb, wv_ref[...], preferred_element_type=jnp.float32) + bv_ref[...]
    s_ref[...] = xb + jnp.dot(xb, ws_ref[...], preferred_element_type=jnp.float32) + bs_ref[...]


def _edge_body(nper, nsub, H, D,
               q_hbm, k_hbm, v_hbm, skip_hbm, es_hbm, ed_hbm, rp_hbm, out_hbm,
               agg, den, outb, skipb, rpbv, esb, edb, edx, qrows, krows,
               vrows, dotbuf, semi0, semi1, semi2, semi3, semr0, semr1):
    HD = H * D
    TD = D // _L
    cid = lax.axis_index("c")
    sid = lax.axis_index("s")
    wid = sid * _NC + cid
    nbase = wid * nper
    inv_sqrt = 1.0 / float(D) ** 0.5
    lane = lax.iota(jnp.int32, _L)
    zeros16 = jnp.zeros((_L,), jnp.float32)
    gidx = [lane * _L + l for l in range(_L)]
    sem_i = (semi0, semi1, semi2, semi3)
    sem_r = (semr0, semr1)

    def sub_body(sub, _):
        m0 = nbase + sub * _SUB

        def z_body(i, _):
            base = i * 128
            for t in range(8):
                agg[pl.ds(base + t * _L, _L)] = zeros16
            return 0
        lax.fori_loop(0, (_SUB * HD) // 128, z_body, 0)

        def zd_body(i, _):
            den[pl.ds(i * _L, _L)] = zeros16
            return 0
        lax.fori_loop(0, _SUB, zd_body, 0)

        pltpu.sync_copy(rp_hbm.at[pl.ds(m0, 4 * _L)], rpbv)
        pltpu.sync_copy(skip_hbm.at[pl.ds(m0 * D, _SUB * D)], skipb)
        e_lo = rpbv[pl.ds(0, _L)][0]
        e_hi = rpbv[pl.ds(_SUB, _L)][0]
        c0 = e_lo // _C
        c1 = (e_hi + (_C - 1)) // _C
        nch = c1 - c0

        def ids_copies(r, bi):
            eb = (c0 + r) * _C
            return (
                (es_hbm.at[pl.ds(eb, 2 * _C)], esb.at[bi], sem_i[bi]),
                (ed_hbm.at[pl.ds(eb, 2 * _C)], edb.at[bi], sem_i[bi]),
                (ed_hbm.at[pl.ds(eb, 2 * _C + _L)], edx.at[bi], sem_i[bi]),
            )

        def rows_copies(bi, br):
            return (
                (q_hbm.at[edb.at[bi, pl.ds(0, _C)]], qrows.at[br], sem_r[br]),
                (k_hbm.at[esb.at[bi, pl.ds(0, _C)]], krows.at[br], sem_r[br]),
                (v_hbm.at[esb.at[bi, pl.ds(0, _C)]], vrows.at[br], sem_r[br]),
            )

        def issue_ids(r, bi):
            @pl.when(r < nch)
            def _():
                for args in ids_copies(r, bi):
                    pltpu.async_copy(*args)

        def wait_ids(r, bi):
            @pl.when(r < nch)
            def _():
                for args in ids_copies(r, bi):
                    pltpu.make_async_copy(*args).wait()

        def issue_rows(r, bi, br):
            @pl.when(r < nch)
            def _():
                for args in rows_copies(bi, br):
                    pltpu.async_copy(*args)

        def wait_rows(r, bi, br):
            @pl.when(r < nch)
            def _():
                for args in rows_copies(bi, br):
                    pltpu.make_async_copy(*args).wait()

        def compute(r, bi, br):
            @pl.when(r < nch)
            def _():
                ebase = (c0 + r) * _C

                def edge_body(j, _):
                    ej = ebase + j
                    ok = (ej >= e_lo) & (ej < e_hi)

                    @pl.when(ok)
                    def _():
                        d_e = edx[bi, pl.ds(j, _L)][0]
                        local = d_e - m0
                        for h in range(H):
                            acc = zeros16
                            for t in range(TD):
                                off = h * D + t * _L
                                acc = acc + (qrows[br, j, pl.ds(off, _L)]
                                             * krows[br, j, pl.ds(off, _L)])
                            dotbuf[pl.ds(h * _L, _L)] = acc
                        lvec = zeros16
                        for l in range(_L):
                            lvec = lvec + plsc.load_gather(dotbuf, [gidx[l]])
                        ex = jnp.exp(lvec * inv_sqrt)
                        ex = jnp.where(lane < H, ex, 0.0)
                        plsc.addupdate(den.at[pl.ds(local * _L, _L)], ex)
                        for h in range(H):
                            eh = ex[h]
                            for t in range(TD):
                                off = h * D + t * _L
                                plsc.addupdate(
                                    agg.at[pl.ds(local * HD + off, _L)],
                                    eh * vrows[br, j, pl.ds(off, _L)])
                    return 0
                lax.fori_loop(0, _C, edge_body, 0)

        issue_ids(0, 0)
        wait_ids(0, 0)
        issue_rows(0, 0, 0)
        issue_ids(1, 1)

        def quad_body(g4, _):
            for b in range(4):
                r = 4 * g4 + b
                wait_ids(r + 1, (b + 1) % 4)
                issue_rows(r + 1, (b + 1) % 4, (b + 1) % 2)
                wait_rows(r, b, b % 2)
                issue_ids(r + 2, (b + 2) % 4)
                compute(r, b, b % 2)
            return 0
        lax.fori_loop(0, (nch + 3) // 4, quad_body, 0)

        def fin_body(n, _):
            dvec = den[pl.ds(n * _L, _L)]
            inv = (1.0 / float(H)) / (dvec + 1e-16)
            for t in range(TD):
                o = skipb[pl.ds(n * D + t * _L, _L)]
                for h in range(H):
                    o = o + inv[h] * agg[pl.ds(n * HD + h * D + t * _L, _L)]
                outb[pl.ds(n * D + t * _L, _L)] = o
            return 0
        lax.fori_loop(0, _SUB, fin_body, 0)
        pltpu.sync_copy(outb, out_hbm.at[pl.ds(m0 * D, _SUB * D)])
        return 0
    lax.fori_loop(0, nsub, sub_body, 0)


def kernel(x, ei, Wq, bq, Wk, bk, Wv, bv, Ws, bs):
    N, D = x.shape
    HD = Wq.shape[1]
    H = HD // D
    E = ei.shape[1]
    nper = _cdiv(N, _NW * _SUB) * _SUB
    npad = nper * _NW
    nsub = nper // _SUB

    xp = jnp.pad(x, ((0, npad - N), (0, 0)))
    bn = 1280
    while npad % bn != 0:
        bn //= 2
    q_p, k_p, v_p, skip_p = pl.pallas_call(
        _proj_body,
        grid=(npad // bn,),
        in_specs=[
            pl.BlockSpec((bn, D), lambda i: (i, 0)),
            pl.BlockSpec((D, HD), lambda i: (0, 0)),
            pl.BlockSpec((1, HD), lambda i: (0, 0)),
            pl.BlockSpec((D, HD), lambda i: (0, 0)),
            pl.BlockSpec((1, HD), lambda i: (0, 0)),
            pl.BlockSpec((D, HD), lambda i: (0, 0)),
            pl.BlockSpec((1, HD), lambda i: (0, 0)),
            pl.BlockSpec((D, D), lambda i: (0, 0)),
            pl.BlockSpec((1, D), lambda i: (0, 0)),
        ],
        out_specs=[
            pl.BlockSpec((bn, HD), lambda i: (i, 0)),
            pl.BlockSpec((bn, HD), lambda i: (i, 0)),
            pl.BlockSpec((bn, HD), lambda i: (i, 0)),
            pl.BlockSpec((bn, D), lambda i: (i, 0)),
        ],
        out_shape=[
            jax.ShapeDtypeStruct((npad, HD), jnp.float32),
            jax.ShapeDtypeStruct((npad, HD), jnp.float32),
            jax.ShapeDtypeStruct((npad, HD), jnp.float32),
            jax.ShapeDtypeStruct((npad, D), jnp.float32),
        ],
    )(xp, Wq, bq.reshape(1, HD), Wk, bk.reshape(1, HD),
      Wv, bv.reshape(1, HD), Ws, bs.reshape(1, D))

    src = ei[0].astype(jnp.int32)
    dst = ei[1].astype(jnp.int32)
    order = jnp.argsort(dst)
    ed_s = dst[order]
    es_s = src[order]
    ep = _cdiv(E, _C) * _C + 3 * _C
    es_pad = jnp.pad(es_s, (0, ep - E))
    ed_pad = jnp.pad(ed_s, (0, ep - E), constant_values=npad - 1)
    rp = jnp.searchsorted(
        ed_s, jnp.arange(npad + 4 * _L, dtype=jnp.int32), side="left"
    ).astype(jnp.int32)

    mesh = plsc.VectorSubcoreMesh(core_axis_name="c", subcore_axis_name="s",
                                  num_cores=_NC, num_subcores=_NS)
    edge_call = functools.partial(
        pl.kernel,
        out_type=jax.ShapeDtypeStruct((npad * D,), jnp.float32),
        mesh=mesh,
        compiler_params=pltpu.CompilerParams(needs_layout_passes=False),
        scratch_types=[
            pltpu.VMEM((_SUB * HD,), jnp.float32),
            pltpu.VMEM((_SUB * _L,), jnp.float32),
            pltpu.VMEM((_SUB * D,), jnp.float32),
            pltpu.VMEM((_SUB * D,), jnp.float32),
            pltpu.VMEM((4 * _L,), jnp.int32),
            pltpu.VMEM((4, 2 * _C), jnp.int32),
            pltpu.VMEM((4, 2 * _C), jnp.int32),
            pltpu.VMEM((4, 2 * _C + _L), jnp.int32),
            pltpu.VMEM((2, _C, HD), jnp.float32),
            pltpu.VMEM((2, _C, HD), jnp.float32),
            pltpu.VMEM((2, _C, HD), jnp.float32),
            pltpu.VMEM((_L * _L,), jnp.float32),
            pltpu.SemaphoreType.DMA,
            pltpu.SemaphoreType.DMA,
            pltpu.SemaphoreType.DMA,
            pltpu.SemaphoreType.DMA,
            pltpu.SemaphoreType.DMA,
            pltpu.SemaphoreType.DMA,
        ],
    )(functools.partial(_edge_body, nper, nsub, H, D))
    out_flat = edge_call(q_p, k_p, v_p, skip_p.reshape(npad * D),
                         es_pad, ed_pad, rp)
    return out_flat.reshape(npad, D)[:N]
